# baseline (speedup 1.0000x reference)
import functools

import jax
import jax.numpy as jnp
from jax import lax
from jax.experimental import pallas as pl
from jax.experimental.pallas import tpu as pltpu
from jax.experimental.pallas import tpu_sc as plsc

VOCAB = 1000
D = 1000
SEQ = 50
BATCH = 1024
NC = 2
NS = 16
NW = NC * NS

NBLK_TOTAL = D // 8
NBUF = 5
LANES = 16


def _gather_body(xt_hbm, tablet_hbm, out_hbm, xt_v, rows_v, staging_v, sem):
    w = lax.axis_index("s") * NC + lax.axis_index("c")
    nblk = jnp.where(w < NBLK_TOTAL - 96, 4, 3)

    pltpu.sync_copy(xt_hbm, xt_v)

    def load_rows(i, carry):
        d0 = pl.multiple_of(8 * (w + 32 * i), 8)
        pltpu.sync_copy(tablet_hbm.at[pl.ds(d0, 8)],
                        rows_v.at[pl.ds(pl.multiple_of(8 * i, 8), 8)])
        return carry

    lax.fori_loop(0, nblk, load_rows, 0)

    def item(t, i, k):
        q = lax.rem(k, NBUF)
        q8 = pl.multiple_of(q * 8, 8)
        d0 = pl.multiple_of(8 * (w + 32 * i), 8)
        dst = out_hbm.at[t, pl.ds(d0, 8)]
        src = staging_v.at[pl.ds(q8, 8)]

        @pl.when(k >= NBUF)
        def _():
            pltpu.make_async_copy(src, dst, sem).wait()

        rvecs = [jnp.full((LANES,), 1, jnp.int32) * (8 * i + rr)
                 for rr in range(8)]

        @plsc.parallel_loop(0, BATCH // LANES, unroll=8)
        def jbody(j):
            c16 = pl.multiple_of(LANES * j, LANES)
            idxv = xt_v[t, pl.ds(c16, LANES)]
            for rr in range(8):
                v = plsc.load_gather(rows_v, [rvecs[rr], idxv])
                staging_v[q8 + rr, pl.ds(c16, LANES)] = v
        pltpu.make_async_copy(src, dst, sem).start()
        return k + 1

    def tbody(t, k):
        return lax.fori_loop(0, nblk, functools.partial(item, t), k)

    kend = lax.fori_loop(0, SEQ, tbody, 0)

    def drain(_, carry):
        pltpu.make_async_copy(
            staging_v.at[pl.ds(0, 8)], out_hbm.at[0, pl.ds(0, 8)], sem
        ).wait()
        return carry

    lax.fori_loop(0, NBUF, drain, 0)


def _gather(xt, tablet):
    mesh = plsc.VectorSubcoreMesh(
        core_axis_name="c", subcore_axis_name="s", num_cores=NC,
        num_subcores=NS,
    )
    run = pl.kernel(
        _gather_body,
        out_type=jax.ShapeDtypeStruct((SEQ, D, BATCH), jnp.float32),
        mesh=mesh,
        compiler_params=pltpu.CompilerParams(needs_layout_passes=False),
        scratch_types=[
            pltpu.VMEM((SEQ, BATCH), jnp.int32),
            pltpu.VMEM((32, VOCAB), jnp.float32),
            pltpu.VMEM((NBUF * 8, BATCH), jnp.float32),
            pltpu.SemaphoreType.DMA,
        ],
    )
    return run(xt, tablet)


def kernel(x, table):
    xt = x.T
    tablet = table.T
    out_t = _gather(xt, tablet)
    return jnp.transpose(out_t, (2, 0, 1))


# speedup vs baseline: 1.0209x; 1.0209x over previous
import functools

import jax
import jax.numpy as jnp
from jax import lax
from jax.experimental import pallas as pl
from jax.experimental.pallas import tpu as pltpu
from jax.experimental.pallas import tpu_sc as plsc

VOCAB = 1000
D = 1000
SEQ = 50
BATCH = 1024
NC = 2
NS = 16
NW = NC * NS

NBLK_TOTAL = D // 8
NBUF = 5
LANES = 16


def _gather_body(xt_hbm, tablet_hbm, out_hbm, xt_v, rows_v, staging_v, sem,
                 lsem):
    w = lax.axis_index("s") * NC + lax.axis_index("c")
    nblk = jnp.where(w < NBLK_TOTAL - 96, 4, 3)

    pltpu.make_async_copy(xt_hbm, xt_v, lsem).start()

    def load_rows(i, carry):
        d0 = pl.multiple_of(8 * (w + 32 * i), 8)
        pltpu.make_async_copy(tablet_hbm.at[pl.ds(d0, 8)],
                              rows_v.at[pl.ds(pl.multiple_of(8 * i, 8), 8)],
                              lsem).start()
        return carry

    lax.fori_loop(0, nblk, load_rows, 0)

    pltpu.make_async_copy(xt_hbm, xt_v, lsem).wait()

    def drain_rows(i, carry):
        pltpu.make_async_copy(tablet_hbm.at[pl.ds(0, 8)],
                              rows_v.at[pl.ds(0, 8)], lsem).wait()
        return carry

    lax.fori_loop(0, nblk, drain_rows, 0)

    def item(t, i, k):
        q = lax.rem(k, NBUF)
        q8 = pl.multiple_of(q * 8, 8)
        d0 = pl.multiple_of(8 * (w + 32 * i), 8)
        dst = out_hbm.at[t, pl.ds(d0, 8)]
        src = staging_v.at[pl.ds(q8, 8)]

        @pl.when(k >= NBUF)
        def _():
            pltpu.make_async_copy(src, dst, sem).wait()

        rvecs = [jnp.full((LANES,), 1, jnp.int32) * (8 * i + rr)
                 for rr in range(8)]

        @plsc.parallel_loop(0, BATCH // LANES, unroll=8)
        def jbody(j):
            c16 = pl.multiple_of(LANES * j, LANES)
            idxv = xt_v[t, pl.ds(c16, LANES)]
            for rr in range(8):
                v = plsc.load_gather(rows_v, [rvecs[rr], idxv])
                staging_v[q8 + rr, pl.ds(c16, LANES)] = v
        pltpu.make_async_copy(src, dst, sem).start()
        return k + 1

    def tbody(t, k):
        return lax.fori_loop(0, nblk, functools.partial(item, t), k)

    kend = lax.fori_loop(0, SEQ, tbody, 0)

    def drain(_, carry):
        pltpu.make_async_copy(
            staging_v.at[pl.ds(0, 8)], out_hbm.at[0, pl.ds(0, 8)], sem
        ).wait()
        return carry

    lax.fori_loop(0, NBUF, drain, 0)


def _gather(xt, tablet):
    mesh = plsc.VectorSubcoreMesh(
        core_axis_name="c", subcore_axis_name="s", num_cores=NC,
        num_subcores=NS,
    )
    run = pl.kernel(
        _gather_body,
        out_type=jax.ShapeDtypeStruct((SEQ, D, BATCH), jnp.float32),
        mesh=mesh,
        compiler_params=pltpu.CompilerParams(needs_layout_passes=False),
        scratch_types=[
            pltpu.VMEM((SEQ, BATCH), jnp.int32),
            pltpu.VMEM((32, VOCAB), jnp.float32),
            pltpu.VMEM((NBUF * 8, BATCH), jnp.float32),
            pltpu.SemaphoreType.DMA,
            pltpu.SemaphoreType.DMA,
        ],
    )
    return run(xt, tablet)


def kernel(x, table):
    xt = x.T
    tablet = table.T
    out_t = _gather(xt, tablet)
    return jnp.transpose(out_t, (2, 0, 1))
